# Initial kernel scaffold; baseline (speedup 1.0000x reference)
import jax
import jax.numpy as jnp
from jax.experimental import pallas as pl


def kernel(x, edge_index, lift_x_w, lift_x_b, taus, logit_scale, readout_w, readout_b):
    raise NotImplementedError("write your pallas kernel here")


# speedup vs baseline: 4.0125x; 4.0125x over previous
import functools

import jax
import jax.numpy as jnp
from jax import lax
from jax.experimental import pallas as pl
from jax.experimental.pallas import tpu as pltpu
from jax.experimental.pallas import tpu_sc as plsc

N = 10000
E = 320000
H = 128
CLS = 40
LAYERS = 15
DT = 0.1
JACOBI = 2

NS = 16
EPT = E // NS
EC = 80
NCHUNK = EPT // EC
RPT = 624
RC = 16

BLK = 1000


def _spmv2_body(g2, src2, dst, zrows, out2, src_v, dst_v, rows_v, acc, sem):
    c = lax.axis_index("c")
    s = lax.axis_index("s")
    row0 = s * RPT
    nrch = jnp.where(s == NS - 1, (N - (NS - 1) * RPT) // RC, RPT // RC)

    def zbody(j, carry):
        pltpu.sync_copy(zrows, acc.at[pl.ds(row0 + j * RC, RC)])
        return carry

    lax.fori_loop(0, nrch, zbody, 0)
    plsc.subcore_barrier()
    ebase = c * E + s * EPT
    dbase = s * EPT

    def body(k, carry):
        eb = ebase + k * EC
        db = dbase + k * EC
        pltpu.sync_copy(src2.at[pl.ds(eb, EC)], src_v)
        pltpu.sync_copy(dst.at[pl.ds(db, EC)], dst_v)
        pltpu.async_copy(g2.at[src_v], rows_v, sem).wait()
        pltpu.sync_copy(rows_v, acc.at[dst_v], add=True)
        return carry

    lax.fori_loop(0, NCHUNK, body, 0)
    plsc.subcore_barrier()
    ob = c * N + row0

    def wbody(j, carry):
        pltpu.sync_copy(acc.at[pl.ds(row0 + j * RC, RC)],
                        out2.at[pl.ds(ob + j * RC, RC)])
        return carry

    lax.fori_loop(0, nrch, wbody, 0)


_spmv2 = pl.kernel(
    _spmv2_body,
    out_type=jax.ShapeDtypeStruct((2 * N, H), jnp.float32),
    mesh=plsc.VectorSubcoreMesh(core_axis_name="c", subcore_axis_name="s"),
    scratch_types=[
        pltpu.VMEM((EC,), jnp.int32),
        pltpu.VMEM((EC,), jnp.int32),
        pltpu.VMEM((EC, H), jnp.float32),
        pltpu.VMEM_SHARED((N, H), jnp.float32),
        pltpu.SemaphoreType.DMA,
    ],
)


def _lift_body(x_ref, w_ref, b_ref, o_ref):
    o_ref[...] = jnp.tanh(
        jnp.dot(x_ref[...], w_ref[...], preferred_element_type=jnp.float32)
        + b_ref[...])


def _readout_body(x_ref, w_ref, b_ref, o_ref):
    o_ref[...] = (
        jnp.dot(x_ref[...], w_ref[...], preferred_element_type=jnp.float32)
        + b_ref[...])


def _stage_a_body(x_ref, y_ref, d_ref, r2_ref, g2_ref):
    x = x_ref[...]
    y = y_ref[...]
    d = d_ref[...]
    rx = x + DT * (x * (1.0 - y))
    ry = y + DT * (y * (x - 1.0))
    r2_ref[0] = rx
    r2_ref[1] = ry
    g2_ref[0] = d * rx
    g2_ref[1] = d * ry


def _stage_b_body(g2_ref, q2_ref, d_ref, w2_ref):
    d = d_ref[...]
    scale = 1.0 / (1.0 + DT)
    dd = d * d
    w2_ref[0] = (g2_ref[0] + DT * dd * q2_ref[0]) * scale
    w2_ref[1] = (g2_ref[1] + DT * dd * q2_ref[1]) * scale


def _stage_c_body(tau_ref, x_ref, y_ref, r2_ref, q2_ref, d_ref,
                  xo_ref, yo_ref):
    d = d_ref[...]
    tau = tau_ref[0]
    scale = 1.0 / (1.0 + DT)
    zx = (r2_ref[0] + DT * d * q2_ref[0]) * scale
    zy = (r2_ref[1] + DT * d * q2_ref[1]) * scale
    xo_ref[...] = (1.0 - tau) * x_ref[...] + tau * zx
    yo_ref[...] = (1.0 - tau) * y_ref[...] + tau * zy


_GRID = N // BLK
_row_spec = pl.BlockSpec((BLK, H), lambda i: (i, 0))
_pair_spec = pl.BlockSpec((2, BLK, H), lambda i: (0, i, 0))
_full_spec = pl.BlockSpec((H, H), lambda i: (0, 0))
_brow_spec = pl.BlockSpec((1, H), lambda i: (0, 0))

_lift = pl.pallas_call(
    _lift_body,
    grid=(_GRID,),
    in_specs=[_row_spec, _full_spec, _brow_spec],
    out_specs=_row_spec,
    out_shape=jax.ShapeDtypeStruct((N, H), jnp.float32),
)

_readout = pl.pallas_call(
    _readout_body,
    grid=(_GRID,),
    in_specs=[_row_spec, _full_spec, _brow_spec],
    out_specs=_row_spec,
    out_shape=jax.ShapeDtypeStruct((N, H), jnp.float32),
)

_stage_a = pl.pallas_call(
    _stage_a_body,
    grid=(_GRID,),
    in_specs=[_row_spec, _row_spec, _row_spec],
    out_specs=[_pair_spec, _pair_spec],
    out_shape=[jax.ShapeDtypeStruct((2, N, H), jnp.float32),
               jax.ShapeDtypeStruct((2, N, H), jnp.float32)],
)

_stage_b = pl.pallas_call(
    _stage_b_body,
    grid=(_GRID,),
    in_specs=[_pair_spec, _pair_spec, _row_spec],
    out_specs=_pair_spec,
    out_shape=jax.ShapeDtypeStruct((2, N, H), jnp.float32),
)

_stage_c = pl.pallas_call(
    _stage_c_body,
    grid=(_GRID,),
    in_specs=[pl.BlockSpec(memory_space=pltpu.SMEM),
              _row_spec, _row_spec, _pair_spec, _pair_spec, _row_spec],
    out_specs=[_row_spec, _row_spec],
    out_shape=[jax.ShapeDtypeStruct((N, H), jnp.float32),
               jax.ShapeDtypeStruct((N, H), jnp.float32)],
)


def kernel(x, edge_index, lift_x_w, lift_x_b, taus, logit_scale,
           readout_w, readout_b):
    src = edge_index[0]
    dst = edge_index[1]
    src2 = jnp.concatenate([src, src + N])

    ones2 = jnp.ones((2 * N, H), dtype=jnp.float32)
    zrows = jnp.zeros((RC, H), dtype=jnp.float32)
    deg = _spmv2(ones2, src2, dst, zrows)[:N, :1]
    dinv = 1.0 / jnp.sqrt(jnp.maximum(deg, 1.0))
    dinvb = jnp.broadcast_to(dinv, (N, H))

    X = _lift(x, lift_x_w, lift_x_b.reshape(1, H))
    Y = jnp.ones_like(X)

    sig_taus = jax.nn.sigmoid(taus)

    for li in range(LAYERS):
        r2, g2 = _stage_a(X, Y, dinvb)
        q2 = _spmv2(g2.reshape(2 * N, H), src2, dst, zrows)
        w2 = _stage_b(g2, q2.reshape(2, N, H), dinvb)
        q2b = _spmv2(w2.reshape(2 * N, H), src2, dst, zrows)
        X, Y = _stage_c(sig_taus[li].reshape(1), X, Y, r2,
                        q2b.reshape(2, N, H), dinvb)

    wp = jnp.zeros((H, H), jnp.float32).at[:, :CLS].set(
        logit_scale * readout_w)
    bp = jnp.zeros((1, H), jnp.float32).at[0, :CLS].set(readout_b)
    out = _readout(X, wp, bp)
    return out[:, :CLS]


# speedup vs baseline: 7.2761x; 1.8133x over previous
import functools

import jax
import jax.numpy as jnp
from jax import lax
from jax.experimental import pallas as pl
from jax.experimental.pallas import tpu as pltpu
from jax.experimental.pallas import tpu_sc as plsc

N = 10000
E = 320000
H = 128
CLS = 40
LAYERS = 15
DT = 0.1
JACOBI = 2

NS = 16
EPT = E // NS
EC = 80
NCHUNK = EPT // EC
RPT = 624
RC = 16

BLK = 1000


def _spmv2_body(g2, src2, dst, zrows, out2,
                src_a, src_b, dst_a, dst_b, rows_a, rows_b, acc,
                ssem_a, ssem_b, dsem_a, dsem_b, gsem_a, gsem_b):
    c = lax.axis_index("c")
    s = lax.axis_index("s")
    row0 = s * RPT
    nrch = jnp.where(s == NS - 1, (N - (NS - 1) * RPT) // RC, RPT // RC)
    ebase = c * E + s * EPT
    dbase = s * EPT

    def istart(k, src_c, dst_c, ssem, dsem):
        pltpu.make_async_copy(
            src2.at[pl.ds(ebase + k * EC, EC)], src_c, ssem).start()
        pltpu.make_async_copy(
            dst.at[pl.ds(dbase + k * EC, EC)], dst_c, dsem).start()

    def iwait(src_c, dst_c, ssem, dsem):
        pltpu.make_async_copy(src2.at[pl.ds(0, EC)], src_c, ssem).wait()
        pltpu.make_async_copy(dst.at[pl.ds(0, EC)], dst_c, dsem).wait()

    def gstart(src_c, rows, gsem):
        pltpu.make_async_copy(g2.at[src_c], rows, gsem).start()

    def gwait(src_c, rows, gsem):
        pltpu.make_async_copy(g2.at[src_c], rows, gsem).wait()

    istart(0, src_a, dst_a, ssem_a, dsem_a)
    istart(1, src_b, dst_b, ssem_b, dsem_b)

    def zbody(j, carry):
        pltpu.sync_copy(zrows, acc.at[pl.ds(row0 + j * RC, RC)])
        return carry

    lax.fori_loop(0, nrch, zbody, 0)
    plsc.subcore_barrier()

    iwait(src_a, dst_a, ssem_a, dsem_a)
    gstart(src_a, rows_a, gsem_a)
    nhalf = NCHUNK // 2

    def body(i, carry):
        k0 = 2 * i
        k1 = k0 + 1
        gwait(src_a, rows_a, gsem_a)
        iwait(src_b, dst_b, ssem_b, dsem_b)
        gstart(src_b, rows_b, gsem_b)
        pltpu.sync_copy(rows_a, acc.at[dst_a], add=True)

        @pl.when(k0 + 2 < NCHUNK)
        def _():
            istart(k0 + 2, src_a, dst_a, ssem_a, dsem_a)

        gwait(src_b, rows_b, gsem_b)

        @pl.when(k1 + 1 < NCHUNK)
        def _():
            iwait(src_a, dst_a, ssem_a, dsem_a)
            gstart(src_a, rows_a, gsem_a)

        pltpu.sync_copy(rows_b, acc.at[dst_b], add=True)

        @pl.when(k1 + 2 < NCHUNK)
        def _():
            istart(k1 + 2, src_b, dst_b, ssem_b, dsem_b)

        return carry

    lax.fori_loop(0, nhalf, body, 0)
    plsc.subcore_barrier()
    ob = c * N + row0

    def wbody(j, carry):
        pltpu.sync_copy(acc.at[pl.ds(row0 + j * RC, RC)],
                        out2.at[pl.ds(ob + j * RC, RC)])
        return carry

    lax.fori_loop(0, nrch, wbody, 0)


_spmv2 = pl.kernel(
    _spmv2_body,
    out_type=jax.ShapeDtypeStruct((2 * N, H), jnp.float32),
    mesh=plsc.VectorSubcoreMesh(core_axis_name="c", subcore_axis_name="s"),
    scratch_types=[
        pltpu.VMEM((EC,), jnp.int32),
        pltpu.VMEM((EC,), jnp.int32),
        pltpu.VMEM((EC,), jnp.int32),
        pltpu.VMEM((EC,), jnp.int32),
        pltpu.VMEM((EC, H), jnp.float32),
        pltpu.VMEM((EC, H), jnp.float32),
        pltpu.VMEM_SHARED((N, H), jnp.float32),
        pltpu.SemaphoreType.DMA,
        pltpu.SemaphoreType.DMA,
        pltpu.SemaphoreType.DMA,
        pltpu.SemaphoreType.DMA,
        pltpu.SemaphoreType.DMA,
        pltpu.SemaphoreType.DMA,
    ],
)


def _lift_body(x_ref, w_ref, b_ref, o_ref):
    o_ref[...] = jnp.tanh(
        jnp.dot(x_ref[...], w_ref[...], preferred_element_type=jnp.float32)
        + b_ref[...])


def _readout_body(x_ref, w_ref, b_ref, o_ref):
    o_ref[...] = (
        jnp.dot(x_ref[...], w_ref[...], preferred_element_type=jnp.float32)
        + b_ref[...])


def _stage_a_body(x_ref, y_ref, d_ref, r2_ref, g2_ref):
    x = x_ref[...]
    y = y_ref[...]
    d = d_ref[...]
    rx = x + DT * (x * (1.0 - y))
    ry = y + DT * (y * (x - 1.0))
    r2_ref[0] = rx
    r2_ref[1] = ry
    g2_ref[0] = d * rx
    g2_ref[1] = d * ry


def _stage_b_body(g2_ref, q2_ref, d_ref, w2_ref):
    d = d_ref[...]
    scale = 1.0 / (1.0 + DT)
    dd = d * d
    w2_ref[0] = (g2_ref[0] + DT * dd * q2_ref[0]) * scale
    w2_ref[1] = (g2_ref[1] + DT * dd * q2_ref[1]) * scale


def _stage_c_body(tau_ref, x_ref, y_ref, r2_ref, q2_ref, d_ref,
                  xo_ref, yo_ref):
    d = d_ref[...]
    tau = tau_ref[0]
    scale = 1.0 / (1.0 + DT)
    zx = (r2_ref[0] + DT * d * q2_ref[0]) * scale
    zy = (r2_ref[1] + DT * d * q2_ref[1]) * scale
    xo_ref[...] = (1.0 - tau) * x_ref[...] + tau * zx
    yo_ref[...] = (1.0 - tau) * y_ref[...] + tau * zy


_GRID = N // BLK
_row_spec = pl.BlockSpec((BLK, H), lambda i: (i, 0))
_pair_spec = pl.BlockSpec((2, BLK, H), lambda i: (0, i, 0))
_full_spec = pl.BlockSpec((H, H), lambda i: (0, 0))
_brow_spec = pl.BlockSpec((1, H), lambda i: (0, 0))

_lift = pl.pallas_call(
    _lift_body,
    grid=(_GRID,),
    in_specs=[_row_spec, _full_spec, _brow_spec],
    out_specs=_row_spec,
    out_shape=jax.ShapeDtypeStruct((N, H), jnp.float32),
)

_readout = pl.pallas_call(
    _readout_body,
    grid=(_GRID,),
    in_specs=[_row_spec, _full_spec, _brow_spec],
    out_specs=_row_spec,
    out_shape=jax.ShapeDtypeStruct((N, H), jnp.float32),
)

_stage_a = pl.pallas_call(
    _stage_a_body,
    grid=(_GRID,),
    in_specs=[_row_spec, _row_spec, _row_spec],
    out_specs=[_pair_spec, _pair_spec],
    out_shape=[jax.ShapeDtypeStruct((2, N, H), jnp.float32),
               jax.ShapeDtypeStruct((2, N, H), jnp.float32)],
)

_stage_b = pl.pallas_call(
    _stage_b_body,
    grid=(_GRID,),
    in_specs=[_pair_spec, _pair_spec, _row_spec],
    out_specs=_pair_spec,
    out_shape=jax.ShapeDtypeStruct((2, N, H), jnp.float32),
)

_stage_c = pl.pallas_call(
    _stage_c_body,
    grid=(_GRID,),
    in_specs=[pl.BlockSpec(memory_space=pltpu.SMEM),
              _row_spec, _row_spec, _pair_spec, _pair_spec, _row_spec],
    out_specs=[_row_spec, _row_spec],
    out_shape=[jax.ShapeDtypeStruct((N, H), jnp.float32),
               jax.ShapeDtypeStruct((N, H), jnp.float32)],
)


def kernel(x, edge_index, lift_x_w, lift_x_b, taus, logit_scale,
           readout_w, readout_b):
    src = edge_index[0]
    dst = edge_index[1]
    src2 = jnp.concatenate([src, src + N])


    ones2 = jnp.ones((2 * N, H), dtype=jnp.float32)
    zrows = jnp.zeros((RC, H), dtype=jnp.float32)
    deg = _spmv2(ones2, src2, dst, zrows)[:N, :1]
    dinv = 1.0 / jnp.sqrt(jnp.maximum(deg, 1.0))
    dinvb = jnp.broadcast_to(dinv, (N, H))

    X = _lift(x, lift_x_w, lift_x_b.reshape(1, H))
    Y = jnp.ones_like(X)

    sig_taus = jax.nn.sigmoid(taus)

    for li in range(LAYERS):
        r2, g2 = _stage_a(X, Y, dinvb)
        q2 = _spmv2(g2.reshape(2 * N, H), src2, dst, zrows)
        w2 = _stage_b(g2, q2.reshape(2, N, H), dinvb)
        q2b = _spmv2(w2.reshape(2 * N, H), src2, dst, zrows)
        X, Y = _stage_c(sig_taus[li].reshape(1), X, Y, r2,
                        q2b.reshape(2, N, H), dinvb)

    wp = jnp.zeros((H, H), jnp.float32).at[:, :CLS].set(
        logit_scale * readout_w)
    bp = jnp.zeros((1, H), jnp.float32).at[0, :CLS].set(readout_b)
    out = _readout(X, wp, bp)
    return out[:, :CLS]
